# baseline (speedup 1.0000x reference)
import functools

import jax
import jax.numpy as jnp
from jax import lax
from jax.experimental import pallas as pl
from jax.experimental.pallas import tpu as pltpu
from jax.experimental.pallas import tpu_sc as plsc

_N = 524288
_D = 16
_TABLE = 128 ** 3
_MASK = _TABLE - 1

_NC = 2
_NS = 16
_NW = _NC * _NS
_BPW = _N // _NW

_TCT = _TABLE // 128
_OCT = _N // 128

_C = 1024
_NCH = _BPW // _C
_G = _C // 128
_HI = _C // 16
_OC = _C // 128

_P1 = -1640531535
_P2 = 805459861


def _worker_id():
    return lax.axis_index("s") * _NC + lax.axis_index("c")


def _quant(v):
    f = (v + 1.0) * 64.0
    f = jnp.minimum(jnp.maximum(f, 0.0), 127.0)
    return f.astype(jnp.int32)


_EOFF = [(e >> 3) * (_TCT * 1024) + (e & 7) * 128 for e in range(_D)]


def _gather_kernel(x_hbm, y_hbm, z_hbm, raw_hbm, out_hbm,
                   x_v, y_v, z_v, idx0, idx1, dst0, dst1,
                   sg0, sg1, so0, so1):
    base = _worker_id() * _BPW
    idx = [idx0, idx1]
    dst = [dst0, dst1]
    sg = [sg0, sg1]
    so = [so0, so1]

    pltpu.sync_copy(x_hbm.at[pl.ds(base, _BPW)], x_v)
    pltpu.sync_copy(y_hbm.at[pl.ds(base, _BPW)], y_v)
    pltpu.sync_copy(z_hbm.at[pl.ds(base, _BPW)], z_v)

    def drain(buf_par, sem, n):
        for _ in range(n):
            pltpu.make_async_copy(out_hbm.at[pl.ds(0, 16), :],
                                  buf_par.at[:, pl.ds(0, 128)], sem).wait()

    def hash_fire(c, par, first):
        off = c * _C
        for j in range(_HI):
            s = pl.ds(off + j * 16, 16)
            h = (_quant(x_v[s])
                 ^ (_quant(y_v[s]) * _P1)
                 ^ (_quant(z_v[s]) * _P2))
            h = h & _MASK
            a0 = ((h >> 7) * 1024) + (h & 127)
            for e in range(_D):
                idx[par][e, j // 8, pl.ds((j % 8) * 16, 16)] = a0 + _EOFF[e]
        if not first:
            drain(dst[par], so[par], _G)
        for e in range(_D):
            for g in range(_G):
                pltpu.async_copy(
                    raw_hbm.at[idx[par].at[e, g]],
                    dst[par].at[e, pl.ds(g * 128, 128)],
                    sg[par],
                )

    def finish(c, par):
        drain(dst[par], sg[par], _G)
        ct0 = (base + c * _C) // 128
        for r in range(2):
            for oc in range(_OC):
                row0 = (r * _OCT + ct0 + oc) * 8
                pltpu.async_copy(
                    dst[par].at[pl.ds(r * 8, 8), pl.ds(oc * 128, 128)],
                    out_hbm.at[pl.ds(row0, 8), :],
                    so[par],
                )

    hash_fire(0, 0, True)

    def it_body(i, carry):
        a = 2 * i

        @pl.when(i > 0)
        def _():
            hash_fire(a + 1, 1, False)

        @pl.when(i == 0)
        def _():
            hash_fire(a + 1, 1, True)

        finish(a, 0)

        @pl.when(i < _NCH // 2 - 1)
        def _():
            hash_fire(a + 2, 0, False)

        finish(a + 1, 1)
        return carry

    lax.fori_loop(0, _NCH // 2, it_body, 0)
    drain(dst[0], so[0], _G)
    drain(dst[1], so[1], _G)


@jax.jit
def kernel(xyz, table):
    mesh = plsc.VectorSubcoreMesh(core_axis_name="c", subcore_axis_name="s")
    params = pltpu.CompilerParams(
        needs_layout_passes=False, use_tc_tiling_on_sc=False
    )

    gather = functools.partial(
        pl.kernel,
        mesh=mesh,
        out_type=jax.ShapeDtypeStruct((_OCT * 16, 128), jnp.float32),
        scratch_types=[
            pltpu.VMEM((_BPW,), jnp.float32),
            pltpu.VMEM((_BPW,), jnp.float32),
            pltpu.VMEM((_BPW,), jnp.float32),
            pltpu.VMEM((_D, _G, 128), jnp.int32),
            pltpu.VMEM((_D, _G, 128), jnp.int32),
            pltpu.VMEM((_D, _C), jnp.float32),
            pltpu.VMEM((_D, _C), jnp.float32),
            pltpu.SemaphoreType.DMA,
            pltpu.SemaphoreType.DMA,
            pltpu.SemaphoreType.DMA,
            pltpu.SemaphoreType.DMA,
        ],
        compiler_params=params,
    )(_gather_kernel)

    x, y, z = xyz[:, 0], xyz[:, 1], xyz[:, 2]
    tbl_raw = (table.T.reshape(2, 8, _TCT, 128)
               .transpose(0, 2, 1, 3).reshape(_TABLE * _D))
    out2 = gather(x, y, z, tbl_raw)
    return (out2.reshape(2, _OCT, 8, 128)
            .transpose(0, 2, 1, 3).reshape(_D, _N).T)


# speedup vs baseline: 1.0213x; 1.0213x over previous
import functools

import jax
import jax.numpy as jnp
from jax import lax
from jax.experimental import pallas as pl
from jax.experimental.pallas import tpu as pltpu
from jax.experimental.pallas import tpu_sc as plsc

_N = 524288
_D = 16
_TABLE = 128 ** 3
_MASK = _TABLE - 1

_NC = 2
_NS = 16
_NW = _NC * _NS
_BPW = _N // _NW

_TCT = _TABLE // 128
_OCT = _N // 128

_C = 512
_NCH = _BPW // _C
_G = _C // 128
_HI = _C // 16
_OC = _C // 128

_P1 = -1640531535
_P2 = 805459861


def _worker_id():
    return lax.axis_index("s") * _NC + lax.axis_index("c")


def _quant(v):
    f = (v + 1.0) * 64.0
    f = jnp.minimum(jnp.maximum(f, 0.0), 127.0)
    return f.astype(jnp.int32)


_EOFF = [(e >> 3) * (_TCT * 1024) + (e & 7) * 128 for e in range(_D)]


def _gather_kernel(x_hbm, y_hbm, z_hbm, raw_hbm, out_hbm,
                   x_v, y_v, z_v, idx0, idx1, dst0, dst1,
                   sg0, sg1, so0, so1):
    base = _worker_id() * _BPW
    idx = [idx0, idx1]
    dst = [dst0, dst1]
    sg = [sg0, sg1]
    so = [so0, so1]

    pltpu.sync_copy(x_hbm.at[pl.ds(base, _BPW)], x_v)
    pltpu.sync_copy(y_hbm.at[pl.ds(base, _BPW)], y_v)
    pltpu.sync_copy(z_hbm.at[pl.ds(base, _BPW)], z_v)

    def drain(buf_par, sem, n):
        for _ in range(n):
            pltpu.make_async_copy(out_hbm.at[pl.ds(0, 16), :],
                                  buf_par.at[:, pl.ds(0, 128)], sem).wait()

    def hash_fire(c, par, first):
        off = c * _C
        for j in range(_HI):
            s = pl.ds(off + j * 16, 16)
            h = (_quant(x_v[s])
                 ^ (_quant(y_v[s]) * _P1)
                 ^ (_quant(z_v[s]) * _P2))
            h = h & _MASK
            a0 = ((h >> 7) * 1024) + (h & 127)
            for e in range(_D):
                idx[par][e, j // 8, pl.ds((j % 8) * 16, 16)] = a0 + _EOFF[e]
        if not first:
            drain(dst[par], so[par], 4)
        for e in range(_D):
            for g in range(_G):
                pltpu.async_copy(
                    raw_hbm.at[idx[par].at[e, g]],
                    dst[par].at[e, pl.ds(g * 128, 128)],
                    sg[par],
                )

    def finish(c, par):
        drain(dst[par], sg[par], 4)
        ct0 = (base + c * _C) // 128
        for r in range(2):
            for oc in range(_OC):
                row0 = (r * _OCT + ct0 + oc) * 8
                pltpu.async_copy(
                    dst[par].at[pl.ds(r * 8, 8), pl.ds(oc * 128, 128)],
                    out_hbm.at[pl.ds(row0, 8), :],
                    so[par],
                )

    hash_fire(0, 0, True)

    def it_body(i, carry):
        a = 2 * i

        @pl.when(i > 0)
        def _():
            hash_fire(a + 1, 1, False)

        @pl.when(i == 0)
        def _():
            hash_fire(a + 1, 1, True)

        finish(a, 0)

        @pl.when(i < _NCH // 2 - 1)
        def _():
            hash_fire(a + 2, 0, False)

        finish(a + 1, 1)
        return carry

    lax.fori_loop(0, _NCH // 2, it_body, 0)
    drain(dst[0], so[0], 4)
    drain(dst[1], so[1], 4)


@jax.jit
def kernel(xyz, table):
    mesh = plsc.VectorSubcoreMesh(core_axis_name="c", subcore_axis_name="s")
    params = pltpu.CompilerParams(
        needs_layout_passes=False, use_tc_tiling_on_sc=False
    )

    gather = functools.partial(
        pl.kernel,
        mesh=mesh,
        out_type=jax.ShapeDtypeStruct((_OCT * 16, 128), jnp.float32),
        scratch_types=[
            pltpu.VMEM((_BPW,), jnp.float32),
            pltpu.VMEM((_BPW,), jnp.float32),
            pltpu.VMEM((_BPW,), jnp.float32),
            pltpu.VMEM((_D, _G, 128), jnp.int32),
            pltpu.VMEM((_D, _G, 128), jnp.int32),
            pltpu.VMEM((_D, _C), jnp.float32),
            pltpu.VMEM((_D, _C), jnp.float32),
            pltpu.SemaphoreType.DMA,
            pltpu.SemaphoreType.DMA,
            pltpu.SemaphoreType.DMA,
            pltpu.SemaphoreType.DMA,
        ],
        compiler_params=params,
    )(_gather_kernel)

    x, y, z = xyz[:, 0], xyz[:, 1], xyz[:, 2]
    tbl_raw = (table.T.reshape(2, 8, _TCT, 128)
               .transpose(0, 2, 1, 3).reshape(_TABLE * _D))
    out2 = gather(x, y, z, tbl_raw)
    return (out2.reshape(2, _OCT, 8, 128)
            .transpose(0, 2, 1, 3).reshape(_D, _N).T)


# speedup vs baseline: 1.0266x; 1.0051x over previous
import functools

import jax
import jax.numpy as jnp
from jax import lax
from jax.experimental import pallas as pl
from jax.experimental.pallas import tpu as pltpu
from jax.experimental.pallas import tpu_sc as plsc

_N = 524288
_D = 16
_TABLE = 128 ** 3
_MASK = _TABLE - 1

_NC = 2
_NS = 16
_NW = _NC * _NS
_BPW = _N // _NW

_TCT = _TABLE // 128
_OCT = _N // 128

_C = 512
_NCH = _BPW // _C
_G = _C // 128
_HI = _C // 16
_OC = _C // 128

_P1 = -1640531535
_P2 = 805459861


def _worker_id():
    return lax.axis_index("s") * _NC + lax.axis_index("c")


def _quant(v):
    f = (v + 1.0) * 64.0
    f = jnp.minimum(jnp.maximum(f, 0.0), 127.0)
    return f.astype(jnp.int32)


_EOFF = [(e >> 3) * (_TCT * 1024) + (e & 7) * 128 for e in range(_D)]


def _gather_kernel(x_hbm, y_hbm, z_hbm, raw_hbm, out_hbm,
                   x_v, y_v, z_v, idx0, idx1, dst0, dst1,
                   sg0, sg1, so0, so1):
    base = _worker_id() * _BPW
    idx = [idx0, idx1]
    dst = [dst0, dst1]
    sg = [sg0, sg1]
    so = [so0, so1]

    pltpu.sync_copy(x_hbm.at[pl.ds(base, _BPW)], x_v)
    pltpu.sync_copy(y_hbm.at[pl.ds(base, _BPW)], y_v)
    pltpu.sync_copy(z_hbm.at[pl.ds(base, _BPW)], z_v)

    def drain(buf_par, sem, n):
        for _ in range(n):
            pltpu.make_async_copy(out_hbm.at[pl.ds(0, 16), :],
                                  buf_par.at[:, pl.ds(0, 128)], sem).wait()

    def hash_fire(c, par, first):
        off = c * _C
        for j in range(_HI):
            s = pl.ds(off + j * 16, 16)
            h = (_quant(x_v[s])
                 ^ (_quant(y_v[s]) * _P1)
                 ^ (_quant(z_v[s]) * _P2))
            h = h & _MASK
            idx[par][j // 8, pl.ds((j % 8) * 16, 16)] = (
                ((h >> 7) * 1024) + (h & 127))
        if not first:
            drain(dst[par], so[par], 4)
        for e in range(_D):
            view = raw_hbm.at[pl.ds(_EOFF[e], _TABLE * _D - _EOFF[e])]
            for g in range(_G):
                pltpu.async_copy(
                    view.at[idx[par].at[g]],
                    dst[par].at[e, pl.ds(g * 128, 128)],
                    sg[par],
                )

    def finish(c, par):
        drain(dst[par], sg[par], 4)
        ct0 = (base + c * _C) // 128
        for r in range(2):
            for oc in range(_OC):
                row0 = (r * _OCT + ct0 + oc) * 8
                pltpu.async_copy(
                    dst[par].at[pl.ds(r * 8, 8), pl.ds(oc * 128, 128)],
                    out_hbm.at[pl.ds(row0, 8), :],
                    so[par],
                )

    hash_fire(0, 0, True)

    def it_body(i, carry):
        a = 2 * i

        @pl.when(i > 0)
        def _():
            hash_fire(a + 1, 1, False)

        @pl.when(i == 0)
        def _():
            hash_fire(a + 1, 1, True)

        finish(a, 0)

        @pl.when(i < _NCH // 2 - 1)
        def _():
            hash_fire(a + 2, 0, False)

        finish(a + 1, 1)
        return carry

    lax.fori_loop(0, _NCH // 2, it_body, 0)
    drain(dst[0], so[0], 4)
    drain(dst[1], so[1], 4)


@jax.jit
def kernel(xyz, table):
    mesh = plsc.VectorSubcoreMesh(core_axis_name="c", subcore_axis_name="s")
    params = pltpu.CompilerParams(
        needs_layout_passes=False, use_tc_tiling_on_sc=False
    )

    gather = functools.partial(
        pl.kernel,
        mesh=mesh,
        out_type=jax.ShapeDtypeStruct((_OCT * 16, 128), jnp.float32),
        scratch_types=[
            pltpu.VMEM((_BPW,), jnp.float32),
            pltpu.VMEM((_BPW,), jnp.float32),
            pltpu.VMEM((_BPW,), jnp.float32),
            pltpu.VMEM((_G, 128), jnp.int32),
            pltpu.VMEM((_G, 128), jnp.int32),
            pltpu.VMEM((_D, _C), jnp.float32),
            pltpu.VMEM((_D, _C), jnp.float32),
            pltpu.SemaphoreType.DMA,
            pltpu.SemaphoreType.DMA,
            pltpu.SemaphoreType.DMA,
            pltpu.SemaphoreType.DMA,
        ],
        compiler_params=params,
    )(_gather_kernel)

    x, y, z = xyz[:, 0], xyz[:, 1], xyz[:, 2]
    tbl_raw = (table.T.reshape(2, 8, _TCT, 128)
               .transpose(0, 2, 1, 3).reshape(_TABLE * _D))
    out2 = gather(x, y, z, tbl_raw)
    return (out2.reshape(2, _OCT, 8, 128)
            .transpose(0, 2, 1, 3).reshape(_D, _N).T)
